# Initial kernel scaffold; baseline (speedup 1.0000x reference)
import jax
import jax.numpy as jnp
from jax.experimental import pallas as pl


def kernel(feat_p3, feat_p4, feat_p5, stem_w, stem_b, obj_w, obj_b, box_w, box_b):
    raise NotImplementedError("write your pallas kernel here")


# speedup vs baseline: 13.2610x; 13.2610x over previous
import functools
import math

import jax
import jax.numpy as jnp
from jax.experimental import pallas as pl
from jax.experimental.pallas import tpu as pltpu

IMG = 512.0
ASPECT_RATIOS = (0.5, 1.0, 2.0)
STRIDE_SCALE = 8
PRE_NMS_TOPK = 400
POST_NMS_TOPK = 100
NMS_THRESH = 0.7
SCALE_CLAMP = math.log(224.0 / 8.0)
C = 256
NEG = -1e30

_INTERPRET = False

_STEM_PREC = jax.lax.Precision.DEFAULT
_EXACT_PREC = jax.lax.Precision.HIGHEST


def _rpn_level_kernel(xp_ref, w9_ref, sb_ref, ow_ref, ob_ref, bw_ref, bb_ref,
                      out_ref, iou_ref, *, H, W, stride):
    Wp = W + 2
    M = H * Wp
    f32 = jnp.float32

    acc = None
    for ky in range(3):
        for kx in range(3):
            off = ky * Wp + kx
            part = jax.lax.dot_general(
                w9_ref[ky * 3 + kx], xp_ref[:, off:off + M],
                (((1,), (0,)), ((), ())),
                preferred_element_type=f32, precision=_STEM_PREC)
            acc = part if acc is None else acc + part
    stem = jnp.maximum(acc + sb_ref[:, 0:1], 0.0)

    lt = jax.lax.dot_general(ow_ref[...], stem, (((1,), (0,)), ((), ())),
                             preferred_element_type=f32,
                             precision=_STEM_PREC) + ob_ref[:, 0:1]
    dt = jax.lax.dot_general(bw_ref[...], stem, (((1,), (0,)), ((), ())),
                             preferred_element_type=f32,
                             precision=_STEM_PREC) + bb_ref[:, 0:1]

    p_iota = jax.lax.broadcasted_iota(jnp.int32, (3, M), 1)
    a_iota = jax.lax.broadcasted_iota(jnp.int32, (3, M), 0)
    ppos_y = p_iota // Wp
    ppos_x = p_iota - ppos_y * Wp
    valid = ppos_x < W
    key2d = ((ppos_y * W + ppos_x) * 3 + a_iota).astype(f32)
    vals = jnp.where(valid, lt, NEG)

    rows400 = jax.lax.broadcasted_iota(jnp.int32, (PRE_NMS_TOPK, 1), 0)

    def extract(j, carry):
        vals, keys = carry
        m = jnp.max(vals)
        selkey = jnp.min(jnp.where(vals == m, key2d, 3.0e7))
        vals = jnp.where(key2d == selkey, NEG, vals)
        keys = jnp.where(rows400 == j, selkey, keys)
        return vals, keys

    zeros400 = jnp.zeros((PRE_NMS_TOPK, 1), f32)
    _, keys_c = jax.lax.fori_loop(
        0, PRE_NMS_TOPK, extract, (vals, zeros400))

    r400 = jax.lax.broadcasted_iota(jnp.int32, (PRE_NMS_TOPK, PRE_NMS_TOPK), 0)
    c400 = jax.lax.broadcasted_iota(jnp.int32, (PRE_NMS_TOPK, PRE_NMS_TOPK), 1)

    keys = jnp.transpose(keys_c, (1, 0))

    loc = jnp.floor((keys + 0.5) * (1.0 / 3.0))
    a_idx = keys - 3.0 * loc
    y_idx = jnp.floor(loc * (1.0 / W))
    x_idx = loc - W * y_idx

    gx = (x_idx + 0.5) * stride
    gy = (y_idx + 0.5) * stride
    area = float(STRIDE_SCALE * stride) ** 2
    anc = []
    for ar in ASPECT_RATIOS:
        w_a = math.sqrt(area / ar)
        h_a = area / w_a
        anc.append((gx - f32(w_a / 2), gy - f32(h_a / 2),
                    gx + f32(w_a / 2), gy + f32(h_a / 2)))

    def sel_a(v0, v1, v2):
        return jnp.where(a_idx == 0.0, v0, jnp.where(a_idx == 1.0, v1, v2))

    ax1 = sel_a(anc[0][0], anc[1][0], anc[2][0])
    ay1 = sel_a(anc[0][1], anc[1][1], anc[2][1])
    ax2 = sel_a(anc[0][2], anc[1][2], anc[2][2])
    ay2 = sel_a(anc[0][3], anc[1][3], anc[2][3])

    pcol = loc + 2.0 * y_idx
    m_iota = jax.lax.broadcasted_iota(jnp.int32, (M, PRE_NMS_TOPK), 0)
    onehot = (m_iota.astype(f32) == pcol).astype(f32)
    g = jax.lax.dot_general(dt, onehot, (((1,), (0,)), ((), ())),
                            preferred_element_type=f32,
                            precision=_EXACT_PREC)

    def sel_d(d):
        return sel_a(g[4 * 0 + d:4 * 0 + d + 1, :],
                     g[4 * 1 + d:4 * 1 + d + 1, :],
                     g[4 * 2 + d:4 * 2 + d + 1, :])

    d0, d1, d2, d3 = sel_d(0), sel_d(1), sel_d(2), sel_d(3)

    dw = jnp.minimum(d2, SCALE_CLAMP)
    dh = jnp.minimum(d3, SCALE_CLAMP)
    aw = ax2 - ax1
    ah = ay2 - ay1
    acx = ax1 + 0.5 * aw
    acy = ay1 + 0.5 * ah
    pcx = d0 * aw + acx
    pcy = d1 * ah + acy
    pw = jnp.exp(dw) * aw
    ph = jnp.exp(dh) * ah
    bx1 = jnp.clip(pcx - 0.5 * pw, 0.0, IMG)
    by1 = jnp.clip(pcy - 0.5 * ph, 0.0, IMG)
    bx2 = jnp.clip(pcx + 0.5 * pw, 0.0, IMG)
    by2 = jnp.clip(pcy + 0.5 * ph, 0.0, IMG)
    b4 = jnp.concatenate([bx1, by1, bx2, by2], axis=0)
    bt = jnp.transpose(b4, (1, 0))

    x1c, y1c = bt[:, 0:1], bt[:, 1:2]
    x2c, y2c = bt[:, 2:3], bt[:, 3:4]
    area_c = (x2c - x1c) * (y2c - y1c)
    area_r = (b4[2:3, :] - b4[0:1, :]) * (b4[3:4, :] - b4[1:2, :])
    xx1 = jnp.maximum(x1c, b4[0:1, :])
    yy1 = jnp.maximum(y1c, b4[1:2, :])
    xx2 = jnp.minimum(x2c, b4[2:3, :])
    yy2 = jnp.minimum(y2c, b4[3:4, :])
    inter = jnp.clip(xx2 - xx1, 0.0) * jnp.clip(yy2 - yy1, 0.0)
    union = area_c + area_r - inter
    iou_ref[...] = inter / jnp.clip(union, 1e-8)

    lane400 = jax.lax.broadcasted_iota(jnp.int32, (1, PRE_NMS_TOPK), 1)

    def nms_body(i, keep):
        row = iou_ref[pl.ds(i, 1), :]
        keep_i = jnp.sum(keep * (lane400 == i).astype(f32))
        sup = ((row > NMS_THRESH) & (lane400 > i)).astype(f32) * \
            jnp.minimum(keep_i, 1.0)
        return keep * (1.0 - sup)

    keep = jax.lax.fori_loop(0, PRE_NMS_TOPK,
                             nms_body, jnp.ones((1, PRE_NMS_TOPK), f32))

    lower_tri = (r400 <= c400).astype(f32)
    kc = jax.lax.dot_general(keep, lower_tri, (((1,), (0,)), ((), ())),
                             preferred_element_type=f32,
                             precision=_EXACT_PREC)
    notk = 1.0 - keep
    nc = jax.lax.dot_general(notk, lower_tri, (((1,), (0,)), ((), ())),
                             preferred_element_type=f32,
                             precision=_EXACT_PREC)
    nk_total = jnp.sum(keep)
    rank = keep * (kc - 1.0) + notk * (nk_total + nc - 1.0)
    rows100 = jax.lax.broadcasted_iota(jnp.int32, (POST_NMS_TOPK, 1), 0)
    sel = (rows100.astype(f32) == rank).astype(f32)
    out_ref[...] = jax.lax.dot_general(
        sel, bt, (((1,), (0,)), ((), ())),
        preferred_element_type=f32, precision=_EXACT_PREC)


def _rpn_level(xp, w9, sb, ow, ob, bw, bb, *, H, W, stride):
    B = xp.shape[0]
    Mp = (H + 3) * (W + 2)
    kfn = functools.partial(_rpn_level_kernel, H=H, W=W, stride=stride)
    return pl.pallas_call(
        kfn,
        grid=(B,),
        in_specs=[
            pl.BlockSpec((None, C, Mp), lambda b: (b, 0, 0)),
            pl.BlockSpec((9, C, C), lambda b: (0, 0, 0)),
            pl.BlockSpec((C, 1), lambda b: (0, 0)),
            pl.BlockSpec((3, C), lambda b: (0, 0)),
            pl.BlockSpec((3, 1), lambda b: (0, 0)),
            pl.BlockSpec((12, C), lambda b: (0, 0)),
            pl.BlockSpec((12, 1), lambda b: (0, 0)),
        ],
        out_specs=pl.BlockSpec((None, POST_NMS_TOPK, 4), lambda b: (b, 0, 0)),
        out_shape=jax.ShapeDtypeStruct((B, POST_NMS_TOPK, 4), jnp.float32),
        scratch_shapes=[pltpu.VMEM((PRE_NMS_TOPK, PRE_NMS_TOPK), jnp.float32)],
        interpret=_INTERPRET,
    )(xp, w9, sb, ow, ob, bw, bb)


def kernel(feat_p3, feat_p4, feat_p5, stem_w, stem_b, obj_w, obj_b,
           box_w, box_b):
    w9 = jnp.stack([stem_w[:, :, ky, kx]
                    for ky in range(3) for kx in range(3)])
    sb = stem_b.reshape(C, 1)
    ow = obj_w.reshape(3, C)
    ob = obj_b.reshape(3, 1)
    bw = box_w.reshape(12, C)
    bb = box_b.reshape(12, 1)

    outs = []
    for f, stride in ((feat_p3, 8), (feat_p4, 16), (feat_p5, 32)):
        B, _, H, W = f.shape
        xp = jnp.pad(f, ((0, 0), (0, 0), (1, 2), (1, 1)))
        xp = xp.reshape(B, C, (H + 3) * (W + 2))
        outs.append(_rpn_level(xp, w9, sb, ow, ob, bw, bb,
                               H=H, W=W, stride=stride))
    return jnp.concatenate(outs, axis=1)
